# Initial kernel scaffold; baseline (speedup 1.0000x reference)
import jax
import jax.numpy as jnp
from jax.experimental import pallas as pl


def kernel(x, table, conv_w, conv_b):
    raise NotImplementedError("write your pallas kernel here")


# speedup vs baseline: 3.4089x; 3.4089x over previous
import functools
import math

import jax
import jax.numpy as jnp
import numpy as np
from jax import lax
from jax.experimental import pallas as pl
from jax.experimental.pallas import tpu as pltpu
from jax.experimental.pallas import tpu_sc as plsc

B, T, F, D = 1024, 50, 26, 64
NC, NS = 2, 16
NW = NC * NS
N = B * T
ROWS_PER_W = N // NW
C = 32
NCHUNKS = ROWS_PER_W // C
IDX_PER_CHUNK = C * F
NG = 7
IDX_PAD = NG * 128
GSIZES = (128, 128, 128, 128, 128, 128, 64)
NV = D // 16


def _pos_encoding(length, d_model):
    position = np.arange(0, length, dtype=np.float32)[:, None]
    div_term = np.exp(
        np.arange(0, d_model, 2, dtype=np.float32) * -(math.log(10000.0) / d_model))
    pe = np.zeros((length, d_model), dtype=np.float32)
    pe[:, 0::2] = np.sin(position * div_term)
    pe[:, 1::2] = np.cos(position * div_term)
    return pe


_PE = _pos_encoding(T, D)


def _body(table, idxs, peb, wv, out, idx_v, rows_v, out_v, pe_v, w_v, sem0, sem1):
    c = lax.axis_index("c")
    s = lax.axis_index("s")
    wid = s * NC + c
    base_row = wid * ROWS_PER_W

    pltpu.sync_copy(peb, pe_v)
    pltpu.sync_copy(wv, w_v)
    w_lo = w_v[pl.ds(0, 16)]
    w_hi = w_v[pl.ds(16, 16)]
    wlist = [w_lo[f] for f in range(16)] + [w_hi[f - 16] for f in range(16, F)]
    sems = (sem0, sem1)

    def issue(chunk, b):
        chunk_row = wid * NCHUNKS + chunk
        pltpu.sync_copy(idxs.at[chunk_row], idx_v.at[b])
        for g, n in enumerate(GSIZES):
            pltpu.async_copy(
                table.at[idx_v.at[b, g, pl.ds(0, n)]],
                rows_v.at[b, pl.ds(g * 128, n), :],
                sems[b])

    def drain(b):
        for g, n in enumerate(GSIZES):
            pltpu.make_async_copy(
                table.at[idx_v.at[b, g, pl.ds(0, n)]],
                rows_v.at[b, pl.ds(g * 128, n), :],
                sems[b]).wait()

    def compute(chunk, b):
        row0 = base_row + chunk * C

        def rbody(i, carry):
            t = lax.rem(row0 + i, T)
            acc0 = [pe_v[t, pl.ds(16 * d, 16)] for d in range(NV)]
            acc1 = [jnp.zeros((16,), jnp.float32) for _ in range(NV)]
            for f in range(F):
                r = i * F + f
                dst = acc0 if f % 2 == 0 else acc1
                for d in range(NV):
                    dst[d] = dst[d] + wlist[f] * rows_v[b, r, pl.ds(16 * d, 16)]
            for d in range(NV):
                out_v[b, i, pl.ds(16 * d, 16)] = acc0[d] + acc1[d]
            return carry

        lax.fori_loop(0, C, rbody, 0)
        pltpu.sync_copy(out_v.at[b], out.at[pl.ds(row0, C), :])

    issue(jnp.int32(0), 0)
    issue(jnp.int32(1), 1)

    def outer(k, carry):
        for b in range(2):
            chunk = 2 * k + b
            drain(b)
            compute(chunk, b)
            nxt = chunk + 2

            @pl.when(nxt < NCHUNKS)
            def _():
                issue(nxt, b)
        return carry

    lax.fori_loop(0, NCHUNKS // 2, outer, 0)


@functools.partial(
    pl.kernel,
    out_type=jax.ShapeDtypeStruct((N, D), jnp.float32),
    mesh=plsc.VectorSubcoreMesh(
        core_axis_name="c", subcore_axis_name="s", num_cores=NC, num_subcores=NS),
    compiler_params=pltpu.CompilerParams(use_tc_tiling_on_sc=False),
    scratch_types=[
        pltpu.VMEM((2, NG, 128), jnp.int32),
        pltpu.VMEM((2, IDX_PER_CHUNK, D), jnp.float32),
        pltpu.VMEM((2, C, D), jnp.float32),
        pltpu.VMEM((T, D), jnp.float32),
        pltpu.VMEM((32,), jnp.float32),
        pltpu.SemaphoreType.DMA,
        pltpu.SemaphoreType.DMA,
    ],
)
def _embed_bag(table, idxs, peb, wv, out, idx_v, rows_v, out_v, pe_v, w_v, s0, s1):
    _body(table, idxs, peb, wv, out, idx_v, rows_v, out_v, pe_v, w_v, s0, s1)


def kernel(x, table, conv_w, conv_b):
    pe = jnp.asarray(_PE)
    idx3 = jnp.pad(
        x.reshape(NW * NCHUNKS, IDX_PER_CHUNK).astype(jnp.int32),
        ((0, 0), (0, IDX_PAD - IDX_PER_CHUNK)),
    ).reshape(NW * NCHUNKS, NG, 128)
    peb = pe + conv_b[0]
    w32 = jnp.zeros((32,), jnp.float32).at[:F].set(conv_w[0, :, 0])
    out = _embed_bag(table, idx3, peb, w32)
    return (out.reshape(B, T, D), pe)


# speedup vs baseline: 3.6232x; 1.0629x over previous
import functools
import math

import jax
import jax.numpy as jnp
import numpy as np
from jax import lax
from jax.experimental import pallas as pl
from jax.experimental.pallas import tpu as pltpu
from jax.experimental.pallas import tpu_sc as plsc

B, T, F, D = 1024, 50, 26, 64
NC, NS = 2, 16
NW = NC * NS
N = B * T
B_PER_W = B // NW
H = 2 * B_PER_W
HC = T // 2
NV = D // 16


def _pos_encoding(length, d_model):
    position = np.arange(0, length, dtype=np.float32)[:, None]
    div_term = np.exp(
        np.arange(0, d_model, 2, dtype=np.float32) * -(math.log(10000.0) / d_model))
    pe = np.zeros((length, d_model), dtype=np.float32)
    pe[:, 0::2] = np.sin(position * div_term)
    pe[:, 1::2] = np.cos(position * div_term)
    return pe


_PE = _pos_encoding(T, D)


def _body(x, table, peb, wv, out, idx_v, rows_v, out_v, pe_v, w_v, sem0, sem1):
    c = lax.axis_index("c")
    s = lax.axis_index("s")
    wid = s * NC + c
    base_b = wid * B_PER_W
    base_row = wid * (B_PER_W * T)

    pltpu.sync_copy(peb, pe_v)
    pltpu.sync_copy(wv, w_v)
    w_lo = w_v[pl.ds(0, 16)]
    w_hi = w_v[pl.ds(16, 16)]
    wlist = [w_lo[f] for f in range(16)] + [w_hi[f - 16] for f in range(16, F)]
    sems = (sem0, sem1)

    def gather_refs(h, p, r):
        bb = lax.div(h, 2)
        slot = lax.rem(bb, 2)
        return (table.at[idx_v.at[slot, p * HC + r]], rows_v.at[p, r])

    def issue(h, p):
        bb = lax.div(h, 2)
        if p == 0:
            pltpu.sync_copy(x.at[base_b + bb], idx_v.at[lax.rem(bb, 2)])
        for r in range(HC):
            src, dst = gather_refs(h, p, r)
            pltpu.async_copy(src, dst, sems[p])

    def drain(h, p):
        for r in range(HC):
            src, dst = gather_refs(h, p, r)
            pltpu.make_async_copy(src, dst, sems[p]).wait()

    def compute(h, p):
        row0 = base_row + h * HC

        def rbody(i, carry):
            t = p * HC + i
            acc0 = [pe_v[t, pl.ds(16 * d, 16)] for d in range(NV)]
            acc1 = [jnp.zeros((16,), jnp.float32) for _ in range(NV)]
            for f in range(F):
                dst = acc0 if f % 2 == 0 else acc1
                for d in range(NV):
                    dst[d] = dst[d] + wlist[f] * rows_v[p, i, f, pl.ds(16 * d, 16)]
            for d in range(NV):
                out_v[p, i, pl.ds(16 * d, 16)] = acc0[d] + acc1[d]
            return carry

        lax.fori_loop(0, HC, rbody, 0)
        pltpu.sync_copy(out_v.at[p], out.at[pl.ds(row0, HC), :])

    issue(jnp.int32(0), 0)
    issue(jnp.int32(1), 1)

    def outer(k, carry):
        for p in range(2):
            h = 2 * k + p
            drain(h, p)
            compute(h, p)
            nxt = h + 2

            @pl.when(nxt < H)
            def _():
                issue(nxt, p)
        return carry

    lax.fori_loop(0, H // 2, outer, 0)


@functools.partial(
    pl.kernel,
    out_type=jax.ShapeDtypeStruct((N, D), jnp.float32),
    mesh=plsc.VectorSubcoreMesh(
        core_axis_name="c", subcore_axis_name="s", num_cores=NC, num_subcores=NS),
    compiler_params=pltpu.CompilerParams(use_tc_tiling_on_sc=False),
    scratch_types=[
        pltpu.VMEM((2, T, F), jnp.int32),
        pltpu.VMEM((2, HC, F, D), jnp.float32),
        pltpu.VMEM((2, HC, D), jnp.float32),
        pltpu.VMEM((T, D), jnp.float32),
        pltpu.VMEM((32,), jnp.float32),
        pltpu.SemaphoreType.DMA,
        pltpu.SemaphoreType.DMA,
    ],
)
def _embed_bag(x, table, peb, wv, out, idx_v, rows_v, out_v, pe_v, w_v, s0, s1):
    _body(x, table, peb, wv, out, idx_v, rows_v, out_v, pe_v, w_v, s0, s1)


def kernel(x, table, conv_w, conv_b):
    pe = jnp.asarray(_PE)
    peb = pe + conv_b[0]
    w32 = jnp.zeros((32,), jnp.float32).at[:F].set(conv_w[0, :, 0])
    out = _embed_bag(x.astype(jnp.int32), table, peb, w32)
    return (out.reshape(B, T, D), pe)


# speedup vs baseline: 3.6387x; 1.0043x over previous
import functools
import math

import jax
import jax.numpy as jnp
import numpy as np
from jax import lax
from jax.experimental import pallas as pl
from jax.experimental.pallas import tpu as pltpu
from jax.experimental.pallas import tpu_sc as plsc

B, T, F, D = 1024, 50, 26, 64
NC, NS = 2, 16
NW = NC * NS
N = B * T
B_PER_W = B // NW
H = 2 * B_PER_W
HC = T // 2
NV = D // 16


def _pos_encoding(length, d_model):
    position = np.arange(0, length, dtype=np.float32)[:, None]
    div_term = np.exp(
        np.arange(0, d_model, 2, dtype=np.float32) * -(math.log(10000.0) / d_model))
    pe = np.zeros((length, d_model), dtype=np.float32)
    pe[:, 0::2] = np.sin(position * div_term)
    pe[:, 1::2] = np.cos(position * div_term)
    return pe


_PE = _pos_encoding(T, D)


def _body(x, table, peb, wv, out, idx_v, rows_v, out_v, pe_v, w_v, sem0, sem1):
    c = lax.axis_index("c")
    s = lax.axis_index("s")
    wid = s * NC + c
    base_b = wid * B_PER_W
    base_row = wid * (B_PER_W * T)

    pltpu.sync_copy(peb, pe_v)
    pltpu.sync_copy(wv, w_v)
    w_lo = w_v[pl.ds(0, 16)]
    w_hi = w_v[pl.ds(16, 16)]
    wlist = [w_lo[f] for f in range(16)] + [w_hi[f - 16] for f in range(16, F)]
    sems = (sem0, sem1)

    def gather_refs(h, p, r):
        bb = lax.div(h, 2)
        slot = lax.rem(bb, 2)
        return (table.at[idx_v.at[slot, p * HC + r]], rows_v.at[p, r])

    def issue(h, p):
        bb = lax.div(h, 2)
        if p == 0:
            pltpu.sync_copy(x.at[base_b + bb], idx_v.at[lax.rem(bb, 2)])
        for r in range(HC):
            src, dst = gather_refs(h, p, r)
            pltpu.async_copy(src, dst, sems[p])

    def drain(h, p):
        for r in range(HC):
            src, dst = gather_refs(h, p, r)
            pltpu.make_async_copy(src, dst, sems[p]).wait()

    def compute(h, p):
        def rbody(i, carry):
            t = p * HC + i
            acc0 = [pe_v[t, pl.ds(16 * d, 16)] for d in range(NV)]
            acc1 = [jnp.zeros((16,), jnp.float32) for _ in range(NV)]
            for f in range(F):
                dst = acc0 if f % 2 == 0 else acc1
                for d in range(NV):
                    dst[d] = dst[d] + wlist[f] * rows_v[p, i, f, pl.ds(16 * d, 16)]
            for d in range(NV):
                out_v[t, pl.ds(16 * d, 16)] = acc0[d] + acc1[d]
            return carry

        lax.fori_loop(0, HC, rbody, 0)
        if p == 1:
            pltpu.sync_copy(out_v, out.at[base_b + lax.div(h, 2)])

    issue(jnp.int32(0), 0)
    issue(jnp.int32(1), 1)

    def outer(k, carry):
        for p in range(2):
            h = 2 * k + p
            drain(h, p)
            compute(h, p)
            nxt = h + 2

            @pl.when(nxt < H)
            def _():
                issue(nxt, p)
        return carry

    lax.fori_loop(0, H // 2, outer, 0)


@functools.partial(
    pl.kernel,
    out_type=jax.ShapeDtypeStruct((B, T, D), jnp.float32),
    mesh=plsc.VectorSubcoreMesh(
        core_axis_name="c", subcore_axis_name="s", num_cores=NC, num_subcores=NS),
    compiler_params=pltpu.CompilerParams(use_tc_tiling_on_sc=False),
    scratch_types=[
        pltpu.VMEM((2, T, F), jnp.int32),
        pltpu.VMEM((2, HC, F, D), jnp.float32),
        pltpu.VMEM((T, D), jnp.float32),
        pltpu.VMEM((T, D), jnp.float32),
        pltpu.VMEM((32,), jnp.float32),
        pltpu.SemaphoreType.DMA,
        pltpu.SemaphoreType.DMA,
    ],
)
def _embed_bag(x, table, peb, wv, out, idx_v, rows_v, out_v, pe_v, w_v, s0, s1):
    _body(x, table, peb, wv, out, idx_v, rows_v, out_v, pe_v, w_v, s0, s1)


def kernel(x, table, conv_w, conv_b):
    pe = jnp.asarray(_PE)
    peb = pe + conv_b[0]
    w32 = jnp.zeros((32,), jnp.float32).at[:F].set(conv_w[0, :, 0])
    out = _embed_bag(x.astype(jnp.int32), table, peb, w32)
    return (out, pe)
